# Initial kernel scaffold; baseline (speedup 1.0000x reference)
import jax
import jax.numpy as jnp
from jax.experimental import pallas as pl


def kernel(x, emb_table, W, b):
    raise NotImplementedError("write your pallas kernel here")


# speedup vs baseline: 3.9335x; 3.9335x over previous
import functools

import jax
import jax.numpy as jnp
from jax import lax
from jax.experimental import pallas as pl
from jax.experimental.pallas import tpu as pltpu
from jax.experimental.pallas import tpu_sc as plsc


def _project_body(emb_ref, w_ref, b_ref, out_ref):
    out_ref[...] = (
        jnp.dot(emb_ref[...], w_ref[...], preferred_element_type=jnp.float32)
        + b_ref[...]
    )


def _project(emb_table, W, b):
    V, _ = emb_table.shape
    Dout = W.shape[1]
    return pl.pallas_call(
        _project_body,
        out_shape=jax.ShapeDtypeStruct((V, Dout), jnp.float32),
    )(emb_table, W, b.reshape(1, Dout))


@functools.lru_cache(maxsize=None)
def _make_gather(V, D, B):
    info = plsc.get_sparse_core_info()
    NC, NS = info.num_cores, info.num_subcores
    NW = NC * NS
    assert B % NW == 0
    b_per_w = B // NW
    CH = 1024
    assert b_per_w % CH == 0
    n_chunks = b_per_w // CH
    mesh = plsc.VectorSubcoreMesh(core_axis_name="c", subcore_axis_name="s")

    @functools.partial(
        pl.kernel,
        mesh=mesh,
        out_type=jax.ShapeDtypeStruct((B, D), jnp.float32),
        scratch_types=[
            pltpu.VMEM((CH,), jnp.int32),
            pltpu.VMEM((CH, D), jnp.float32),
            pltpu.SemaphoreType.DMA,
        ],
        compiler_params=pltpu.CompilerParams(use_tc_tiling_on_sc=False),
    )
    def gather(table_hbm, idx_hbm, out_hbm, idx_v, rows_v, sem):
        wid = lax.axis_index("s") * NC + lax.axis_index("c")
        base = wid * b_per_w

        def step(i, carry):
            off = base + i * CH
            pltpu.sync_copy(idx_hbm.at[pl.ds(off, CH)], idx_v)
            pltpu.async_copy(table_hbm.at[idx_v], rows_v, sem).wait()
            pltpu.sync_copy(rows_v, out_hbm.at[pl.ds(off, CH)])
            return carry

        lax.fori_loop(0, n_chunks, step, 0)

    return gather


def kernel(x, emb_table, W, b):
    Bm, Lx = x.shape
    V = emb_table.shape[0]
    Dout = W.shape[1]
    proj = _project(emb_table, W, b)
    flat = x.reshape(-1).astype(jnp.int32)
    out = _make_gather(V, Dout, flat.shape[0])(proj, flat)
    return out.reshape(Bm, Lx, Dout)


# speedup vs baseline: 3.9590x; 1.0065x over previous
import functools

import jax
import jax.numpy as jnp
from jax import lax
from jax.experimental import pallas as pl
from jax.experimental.pallas import tpu as pltpu
from jax.experimental.pallas import tpu_sc as plsc


def _project_body(emb_ref, w_ref, b_ref, out_ref):
    out_ref[...] = (
        jnp.dot(emb_ref[...], w_ref[...], preferred_element_type=jnp.float32)
        + b_ref[...]
    )


def _project(emb_table, W, b):
    V, _ = emb_table.shape
    Dout = W.shape[1]
    return pl.pallas_call(
        _project_body,
        out_shape=jax.ShapeDtypeStruct((V, Dout), jnp.float32),
    )(emb_table, W, b.reshape(1, Dout))


@functools.lru_cache(maxsize=None)
def _make_gather(V, D, B):
    info = plsc.get_sparse_core_info()
    NC, NS = info.num_cores, info.num_subcores
    NW = NC * NS
    assert B % NW == 0
    b_per_w = B // NW
    CH = 640
    assert b_per_w % CH == 0
    n_chunks = b_per_w // CH
    assert n_chunks % 2 == 0
    mesh = plsc.VectorSubcoreMesh(core_axis_name="c", subcore_axis_name="s")

    @functools.partial(
        pl.kernel,
        mesh=mesh,
        out_type=jax.ShapeDtypeStruct((B, D), jnp.float32),
        scratch_types=[
            pltpu.VMEM((b_per_w,), jnp.int32),
            pltpu.VMEM((CH, D), jnp.float32),
            pltpu.VMEM((CH, D), jnp.float32),
            pltpu.SemaphoreType.DMA,
            pltpu.SemaphoreType.DMA,
            pltpu.SemaphoreType.DMA,
            pltpu.SemaphoreType.DMA,
        ],
        compiler_params=pltpu.CompilerParams(use_tc_tiling_on_sc=False),
    )
    def gather(table_hbm, idx_hbm, out_hbm, idx_all, rows0, rows1, sg0, sg1, so0, so1):
        wid = lax.axis_index("s") * NC + lax.axis_index("c")
        base = wid * b_per_w
        rows = (rows0, rows1)
        sg = (sg0, sg1)
        so = (so0, so1)

        pltpu.sync_copy(idx_hbm.at[pl.ds(base, b_per_w)], idx_all)

        def gather_start(g, buf):
            pltpu.async_copy(
                table_hbm.at[idx_all.at[pl.ds(g * CH, CH)]], rows[buf], sg[buf]
            )

        gather_start(0, 0)

        def pair(i, carry):
            for buf in range(2):
                g = 2 * i + buf
                pltpu.make_async_copy(
                    table_hbm.at[idx_all.at[pl.ds(0, CH)]], rows[buf], sg[buf]
                ).wait()
                pltpu.async_copy(
                    rows[buf], out_hbm.at[pl.ds(base + g * CH, CH)], so[buf]
                )
                @pl.when(g >= 1)
                def _():
                    pltpu.make_async_copy(
                        rows[1 - buf], out_hbm.at[pl.ds(base, CH)], so[1 - buf]
                    ).wait()

                @pl.when(g + 1 < n_chunks)
                def _():
                    gather_start(g + 1, 1 - buf)
            return carry

        lax.fori_loop(0, n_chunks // 2, pair, 0)
        pltpu.make_async_copy(
            rows[(n_chunks - 1) % 2],
            out_hbm.at[pl.ds(base, CH)],
            so[(n_chunks - 1) % 2],
        ).wait()

    return gather


def kernel(x, emb_table, W, b):
    Bm, Lx = x.shape
    V = emb_table.shape[0]
    Dout = W.shape[1]
    proj = _project(emb_table, W, b)
    flat = x.reshape(-1).astype(jnp.int32)
    out = _make_gather(V, Dout, flat.shape[0])(proj, flat)
    return out.reshape(Bm, Lx, Dout)


# speedup vs baseline: 6.7494x; 1.7048x over previous
import functools

import jax
import jax.numpy as jnp
from jax import lax
from jax.experimental import pallas as pl
from jax.experimental.pallas import tpu as pltpu
from jax.experimental.pallas import tpu_sc as plsc


def _project_body(emb_ref, w_ref, b_ref, out_ref):
    pt = lax.dot_general(
        w_ref[...],
        emb_ref[...],
        dimension_numbers=(((0,), (1,)), ((), ())),
        preferred_element_type=jnp.float32,
    )
    out_ref[...] = pt + b_ref[...]


def _project_t(emb_table, W, b):
    V = emb_table.shape[0]
    Dout = W.shape[1]
    return pl.pallas_call(
        _project_body,
        out_shape=jax.ShapeDtypeStruct((Dout, V), jnp.float32),
    )(emb_table, W, b.reshape(Dout, 1))


@functools.lru_cache(maxsize=None)
def _make_gather(V, D, Bm, Lx):
    info = plsc.get_sparse_core_info()
    NC, NS, L = info.num_cores, info.num_subcores, info.num_lanes
    NW = NC * NS
    assert Bm % NW == 0 and L == 16
    b_per_w = Bm // NW
    HB = b_per_w // 2
    assert HB % L == 0
    n_bb = HB // L
    D_UNROLL = 8
    assert D % D_UNROLL == 0
    mesh = plsc.VectorSubcoreMesh(core_axis_name="c", subcore_axis_name="s")

    @functools.partial(
        pl.kernel,
        mesh=mesh,
        out_type=jax.ShapeDtypeStruct((Lx, D, Bm), jnp.float32),
        scratch_types=[
            pltpu.VMEM((D * V,), jnp.float32),
            pltpu.VMEM((b_per_w * Lx,), jnp.int32),
            pltpu.VMEM((D, HB), jnp.float32),
            pltpu.VMEM((D, HB), jnp.float32),
            pltpu.SemaphoreType.DMA,
            pltpu.SemaphoreType.DMA,
        ],
        compiler_params=pltpu.CompilerParams(needs_layout_passes=False),
    )
    def gather(pt_hbm, x_hbm, out_hbm, pt_v, xs_v, ob0, ob1, so0, so1):
        wid = lax.axis_index("s") * NC + lax.axis_index("c")
        b0 = wid * b_per_w
        obs = (ob0, ob1)
        sos = (so0, so1)

        pltpu.sync_copy(pt_hbm, pt_v)
        pltpu.sync_copy(x_hbm.at[pl.ds(b0 * Lx, b_per_w * Lx)], xs_v)

        lane = lax.iota(jnp.int32, L)
        lane_l = lane * Lx

        def l_body(l, carry):
            for half in range(2):
                ob = obs[half]
                sem = sos[half]

                @pl.when(l >= 1)
                def _():
                    pltpu.make_async_copy(
                        ob, out_hbm.at[0, :, pl.ds(0, HB)], sem
                    ).wait()

                def bb_body(bb, carry2):
                    bbase = (half * n_bb + bb) * L
                    idx = plsc.load_gather(xs_v, [lane_l + (bbase * Lx + l)])

                    def d_body(d8, idxd):
                        for u in range(D_UNROLL):
                            ob[d8 * D_UNROLL + u, pl.ds(bb * L, L)] = (
                                plsc.load_gather(pt_v, [idxd])
                            )
                            idxd = idxd + V
                        return idxd

                    lax.fori_loop(0, D // D_UNROLL, d_body, idx, unroll=1)
                    return carry2

                lax.fori_loop(0, n_bb, bb_body, 0)
                pltpu.async_copy(
                    ob, out_hbm.at[l, :, pl.ds(b0 + half * HB, HB)], sem
                )
            return carry

        lax.fori_loop(0, Lx, l_body, 0)
        pltpu.make_async_copy(ob0, out_hbm.at[0, :, pl.ds(0, HB)], so0).wait()
        pltpu.make_async_copy(ob1, out_hbm.at[0, :, pl.ds(0, HB)], so1).wait()

    return gather


def kernel(x, emb_table, W, b):
    Bm, Lx = x.shape
    V = emb_table.shape[0]
    Dout = W.shape[1]
    pt = _project_t(emb_table, W, b).reshape(-1)
    flat = x.reshape(-1).astype(jnp.int32)
    out_t = _make_gather(V, Dout, Bm, Lx)(pt, flat)
    return jnp.transpose(out_t, (2, 0, 1))


# speedup vs baseline: 22.4267x; 3.3228x over previous
import functools

import jax
import jax.numpy as jnp
from jax import lax
from jax.experimental import pallas as pl
from jax.experimental.pallas import tpu as pltpu
from jax.experimental.pallas import tpu_sc as plsc


def _project_body(emb_ref, w_ref, b_ref, out_ref):
    pt = lax.dot_general(
        w_ref[...],
        emb_ref[...],
        dimension_numbers=(((0,), (1,)), ((), ())),
        preferred_element_type=jnp.float32,
    )
    out_ref[...] = pt + b_ref[...]


def _project_t(emb_table, W, b):
    V = emb_table.shape[0]
    Dout = W.shape[1]
    return pl.pallas_call(
        _project_body,
        out_shape=jax.ShapeDtypeStruct((Dout, V), jnp.float32),
    )(emb_table, W, b.reshape(Dout, 1))


@functools.lru_cache(maxsize=None)
def _make_gather(V, D, Bm, Lx):
    info = plsc.get_sparse_core_info()
    NC, NS, L = info.num_cores, info.num_subcores, info.num_lanes
    NW = NC * NS
    assert Bm % NW == 0 and L == 16
    b_per_w = Bm // NW
    HB = b_per_w // 2
    assert HB % L == 0
    n_bb = HB // L
    D_UNROLL = 8
    assert D % D_UNROLL == 0
    mesh = plsc.VectorSubcoreMesh(core_axis_name="c", subcore_axis_name="s")

    @functools.partial(
        pl.kernel,
        mesh=mesh,
        out_type=jax.ShapeDtypeStruct((Lx, D, Bm), jnp.float32),
        scratch_types=[
            pltpu.VMEM((D * V,), jnp.float32),
            pltpu.VMEM((b_per_w * Lx,), jnp.int32),
            pltpu.VMEM((D, HB), jnp.float32),
            pltpu.VMEM((D, HB), jnp.float32),
            pltpu.SemaphoreType.DMA,
            pltpu.SemaphoreType.DMA,
        ],
        compiler_params=pltpu.CompilerParams(needs_layout_passes=False),
    )
    def gather(pt_hbm, x_hbm, out_hbm, pt_v, xs_v, ob0, ob1, so0, so1):
        wid = lax.axis_index("s") * NC + lax.axis_index("c")
        b0 = wid * b_per_w
        obs = (ob0, ob1)
        sos = (so0, so1)

        pltpu.sync_copy(pt_hbm, pt_v)
        pltpu.sync_copy(x_hbm.at[pl.ds(b0 * Lx, b_per_w * Lx)], xs_v)

        lane = lax.iota(jnp.int32, L)
        lane_l = lane * Lx

        def l_body(l, carry):
            for half in range(2):
                ob = obs[half]
                sem = sos[half]

                @pl.when(l >= 1)
                def _():
                    pltpu.make_async_copy(
                        ob, out_hbm.at[0, :, pl.ds(0, HB)], sem
                    ).wait()

                def bb_body(bb):
                    bbase = (half * n_bb + bb) * L
                    idx = plsc.load_gather(xs_v, [lane_l + (bbase * Lx + l)])

                    def d_body(d, idxd):
                        ob[d, pl.ds(bb * L, L)] = plsc.load_gather(pt_v, [idxd])
                        return idxd + V

                    plsc.parallel_loop(0, D, unroll=D_UNROLL, carry=idx)(d_body)

                plsc.parallel_loop(0, n_bb)(bb_body)
                pltpu.async_copy(
                    ob, out_hbm.at[l, :, pl.ds(b0 + half * HB, HB)], sem
                )
            return carry

        lax.fori_loop(0, Lx, l_body, 0)
        pltpu.make_async_copy(ob0, out_hbm.at[0, :, pl.ds(0, HB)], so0).wait()
        pltpu.make_async_copy(ob1, out_hbm.at[0, :, pl.ds(0, HB)], so1).wait()

    return gather


def kernel(x, emb_table, W, b):
    Bm, Lx = x.shape
    V = emb_table.shape[0]
    Dout = W.shape[1]
    pt = _project_t(emb_table, W, b).reshape(-1)
    flat = x.reshape(-1).astype(jnp.int32)
    out_t = _make_gather(V, Dout, Bm, Lx)(pt, flat)
    return jnp.transpose(out_t, (2, 0, 1))
